# baseline (speedup 1.0000x reference)
import functools

import jax
import jax.numpy as jnp
from jax import lax
from jax.experimental import pallas as pl
from jax.experimental.pallas import tpu as pltpu
from jax.experimental.pallas import tpu_sc as plsc

N = 10000
E = 320000
D = 128
H = 128
C = 40
STEP = 0.1
CPAD = 128

ROW_BLK = 1000

NUM_TILES = 32
EPT = E // NUM_TILES
CHUNK = 40
NCHUNK = EPT // CHUNK
NBUF = 4
NPAD = 10240
RPT = NPAD // 16


def _stage_a_body(x_ref, l1w_ref, l1b_ref, w1_ref, w2_ref, wf_ref,
                  q_ref, h3_ref, h4_ref, hf_ref):
    q = jnp.dot(x_ref[...], l1w_ref[...], preferred_element_type=jnp.float32)
    q = jnp.maximum(q + l1b_ref[...], 0.0)
    q_ref[...] = q
    h3_ref[...] = jnp.dot(q, w1_ref[...], preferred_element_type=jnp.float32)
    h4_ref[...] = jnp.dot(q, w2_ref[...], preferred_element_type=jnp.float32)
    hf_ref[...] = jnp.dot(q, wf_ref[...], preferred_element_type=jnp.float32)


def _stage_a(x, l1w_t, l1b, w1_t, w2_t, wf_t):
    grid = (N // ROW_BLK,)
    full = pl.BlockSpec((D, H), lambda i: (0, 0))
    blk = pl.BlockSpec((ROW_BLK, H), lambda i: (i, 0))
    return pl.pallas_call(
        _stage_a_body,
        grid=grid,
        in_specs=[
            pl.BlockSpec((ROW_BLK, D), lambda i: (i, 0)),
            full,
            pl.BlockSpec((1, H), lambda i: (0, 0)),
            full, full, full,
        ],
        out_specs=[blk, blk, blk, blk],
        out_shape=[jax.ShapeDtypeStruct((N, H), jnp.float32)] * 4,
    )(x, l1w_t, l1b, w1_t, w2_t, wf_t)


def _stage_b_body(hf_ref, h4_ref, agg_ref, w1_ref, w2_ref, wf_ref,
                  h3_ref, h4o_ref, hfo_ref):
    g = agg_ref[0, :, :] + agg_ref[1, :, :]
    v = hf_ref[...] + STEP * g - STEP * h4_ref[...]
    nrm = jnp.sqrt(jnp.sum(v * v, axis=1, keepdims=True))
    q = v / jnp.maximum(nrm, 1e-12)
    h3_ref[...] = jnp.dot(q, w1_ref[...], preferred_element_type=jnp.float32)
    h4o_ref[...] = jnp.dot(q, w2_ref[...], preferred_element_type=jnp.float32)
    hfo_ref[...] = jnp.dot(q, wf_ref[...], preferred_element_type=jnp.float32)


def _stage_b(hf, h4, agg, w1_t, w2_t, wf_t):
    grid = (N // ROW_BLK,)
    full = pl.BlockSpec((H, H), lambda i: (0, 0))
    blk = pl.BlockSpec((ROW_BLK, H), lambda i: (i, 0))
    return pl.pallas_call(
        _stage_b_body,
        grid=grid,
        in_specs=[
            blk, blk,
            pl.BlockSpec((2, ROW_BLK, H), lambda i: (0, i, 0)),
            full, full, full,
        ],
        out_specs=[blk, blk, blk],
        out_shape=[jax.ShapeDtypeStruct((N, H), jnp.float32)] * 3,
    )(hf, h4, agg, w1_t, w2_t, wf_t)


def _stage_c_body(hf_ref, h4_ref, agg_ref, q2_ref, wa_ref, wb_ref, b_ref,
                  out_ref):
    g = agg_ref[0, :, :] + agg_ref[1, :, :]
    v = hf_ref[...] + STEP * g - STEP * h4_ref[...]
    nrm = jnp.sqrt(jnp.sum(v * v, axis=1, keepdims=True))
    q3 = v / jnp.maximum(nrm, 1e-12)
    logits = (jnp.dot(q2_ref[...], wa_ref[...], preferred_element_type=jnp.float32)
              + jnp.dot(q3, wb_ref[...], preferred_element_type=jnp.float32)
              + b_ref[...])
    m = jnp.max(logits, axis=1, keepdims=True)
    ex = jnp.exp(logits - m)
    lse = jnp.log(jnp.sum(ex, axis=1, keepdims=True))
    out_ref[...] = logits - m - lse


def _stage_c(hf, h4, agg, q2, wa_t, wb_t, b_pad):
    grid = (N // ROW_BLK,)
    full = pl.BlockSpec((H, CPAD), lambda i: (0, 0))
    blk = pl.BlockSpec((ROW_BLK, H), lambda i: (i, 0))
    return pl.pallas_call(
        _stage_c_body,
        grid=grid,
        in_specs=[
            blk, blk,
            pl.BlockSpec((2, ROW_BLK, H), lambda i: (0, i, 0)),
            blk,
            full, full,
            pl.BlockSpec((1, CPAD), lambda i: (0, 0)),
        ],
        out_specs=pl.BlockSpec((ROW_BLK, CPAD), lambda i: (i, 0)),
        out_shape=jax.ShapeDtypeStruct((N, CPAD), jnp.float32),
    )(hf, h4, agg, q2, wa_t, wb_t, b_pad)


def _edge_body(h3_hbm, h4_hbm, row_hbm, col_hbm, zeros_hbm, out_hbm,
               row_b, col_b, h3r_b, h4r_b, agg_sh, sem_i, sem_g, sem_s):
    c = lax.axis_index("c")
    s = lax.axis_index("s")
    wid = c * 16 + s

    pltpu.sync_copy(zeros_hbm.at[pl.ds(s * RPT, RPT)],
                    agg_sh.at[pl.ds(s * RPT, RPT)])
    plsc.subcore_barrier()

    base = wid * EPT

    def idx_start(j, b):
        off = base + j * CHUNK
        pltpu.async_copy(row_hbm.at[pl.ds(off, CHUNK)], row_b[b], sem_i[b])
        pltpu.async_copy(col_hbm.at[pl.ds(off, CHUNK)], col_b[b], sem_i[b])

    def idx_wait(j, b):
        off = base + j * CHUNK
        pltpu.make_async_copy(row_hbm.at[pl.ds(off, CHUNK)], row_b[b],
                              sem_i[b]).wait()
        pltpu.make_async_copy(col_hbm.at[pl.ds(off, CHUNK)], col_b[b],
                              sem_i[b]).wait()

    def gather_start(b):
        pltpu.async_copy(h3_hbm.at[row_b[b]], h3r_b[b], sem_g[b])
        pltpu.async_copy(h4_hbm.at[col_b[b]], h4r_b[b], sem_g[b])

    def gather_wait(b):
        pltpu.make_async_copy(h3_hbm.at[row_b[b]], h3r_b[b], sem_g[b]).wait()
        pltpu.make_async_copy(h4_hbm.at[col_b[b]], h4r_b[b], sem_g[b]).wait()

    def scatter_wait(b):
        pltpu.make_async_copy(h3r_b[b], agg_sh.at[col_b[b]], sem_s[b]).wait()

    def compute(b):
        h3r_v, h4r_v = h3r_b[b], h4r_b[b]
        gather_wait(b)

        @plsc.parallel_loop(0, CHUNK, unroll=2)
        def edge_body(e):
            v3 = [h3r_v[e, pl.ds(t * 16, 16)] for t in range(8)]
            acc = v3[0] * h4r_v[e, pl.ds(0, 16)]
            for t in range(1, 8):
                acc = acc + v3[t] * h4r_v[e, pl.ds(t * 16, 16)]
            dot = jnp.sum(acc)
            sv = 1.0 / (1.0 + jnp.exp(jnp.broadcast_to(dot, (16,))))
            for t in range(8):
                h3r_v[e, pl.ds(t * 16, 16)] = v3[t] * sv

        pltpu.async_copy(h3r_v, agg_sh.at[col_b[b]], sem_s[b], add=True)

    idx_start(0, 0)
    idx_start(1, 1)
    idx_wait(0, 0)
    gather_start(0)

    def quad_body(p, carry):
        for b in range(NBUF):
            j = NBUF * p + b

            @pl.when(jnp.logical_and(j - 2 >= 0, j - 2 < NCHUNK))
            def _():
                scatter_wait((b - 2) % NBUF)

            @pl.when(j + 2 < NCHUNK)
            def _():
                idx_start(j + 2, (b + 2) % NBUF)

            @pl.when(j + 1 < NCHUNK)
            def _():
                idx_wait(j + 1, (b + 1) % NBUF)
                gather_start((b + 1) % NBUF)

            @pl.when(j < NCHUNK)
            def _():
                compute(b)

        return carry

    lax.fori_loop(0, (NCHUNK + NBUF - 1) // NBUF + 1, quad_body, 0)
    plsc.subcore_barrier()

    pltpu.sync_copy(agg_sh.at[pl.ds(s * RPT, RPT)],
                    out_hbm.at[c, pl.ds(s * RPT, RPT)])


@functools.cache
def _edge_kernel():
    return pl.kernel(
        _edge_body,
        mesh=plsc.VectorSubcoreMesh(core_axis_name="c", subcore_axis_name="s"),
        out_type=jax.ShapeDtypeStruct((2, NPAD, H), jnp.float32),
        scratch_types=[
            [pltpu.VMEM((CHUNK,), jnp.int32)] * NBUF,
            [pltpu.VMEM((CHUNK,), jnp.int32)] * NBUF,
            [pltpu.VMEM((CHUNK, H), jnp.float32)] * NBUF,
            [pltpu.VMEM((CHUNK, H), jnp.float32)] * NBUF,
            pltpu.VMEM_SHARED((NPAD, H), jnp.float32),
            [pltpu.SemaphoreType.DMA] * NBUF,
            [pltpu.SemaphoreType.DMA] * NBUF,
            [pltpu.SemaphoreType.DMA] * NBUF,
        ],
        compiler_params=pltpu.CompilerParams(needs_layout_passes=False),
    )


def kernel(x, edge_index, lin1_W, lin1_b, W1_0, W2_0, Wf_0, W1_1, W2_1, Wf_1,
           out_W, out_b):
    row = edge_index[0]
    col = edge_index[1]
    zeros = jnp.zeros((NPAD, H), jnp.float32)

    q2, h3, h4, hf = _stage_a(
        x, lin1_W.T, lin1_b.reshape(1, H), W1_0.T, W2_0.T, Wf_0.T)
    agg0 = _edge_kernel()(h3, h4, row, col, zeros)
    h3, h4, hf = _stage_b(hf, h4, agg0, W1_1.T, W2_1.T, Wf_1.T)
    agg1 = _edge_kernel()(h3, h4, row, col, zeros)

    wa_t = jnp.zeros((H, CPAD), jnp.float32).at[:, :C].set(out_W[:, :H].T)
    wb_t = jnp.zeros((H, CPAD), jnp.float32).at[:, :C].set(out_W[:, H:].T)
    b_pad = jnp.full((1, CPAD), -1e30, jnp.float32).at[0, :C].set(out_b)

    out_pad = _stage_c(hf, h4, agg1, q2, wa_t, wb_t, b_pad)
    return out_pad[:, :C]


# speedup vs baseline: 1.2413x; 1.2413x over previous
import functools

import jax
import jax.numpy as jnp
from jax import lax
from jax.experimental import pallas as pl
from jax.experimental.pallas import tpu as pltpu
from jax.experimental.pallas import tpu_sc as plsc

N = 10000
E = 320000
D = 128
H = 128
C = 40
STEP = 0.1
CPAD = 128

ROW_BLK = 1000

NUM_TILES = 32
EPT = E // NUM_TILES
CHUNK = 40
NCHUNK = EPT // CHUNK
NBUF = 4
NIDX = 8
NPAD = 10240
RPT = NPAD // 16


def _stage_a_body(x_ref, l1w_ref, l1b_ref, w1_ref, w2_ref, wf_ref,
                  q_ref, h3_ref, h4_ref, hf_ref):
    q = jnp.dot(x_ref[...], l1w_ref[...], preferred_element_type=jnp.float32)
    q = jnp.maximum(q + l1b_ref[...], 0.0)
    q_ref[...] = q
    h3_ref[...] = jnp.dot(q, w1_ref[...], preferred_element_type=jnp.float32)
    h4_ref[...] = jnp.dot(q, w2_ref[...], preferred_element_type=jnp.float32)
    hf_ref[...] = jnp.dot(q, wf_ref[...], preferred_element_type=jnp.float32)


def _stage_a(x, l1w_t, l1b, w1_t, w2_t, wf_t):
    grid = (N // ROW_BLK,)
    full = pl.BlockSpec((D, H), lambda i: (0, 0))
    blk = pl.BlockSpec((ROW_BLK, H), lambda i: (i, 0))
    return pl.pallas_call(
        _stage_a_body,
        grid=grid,
        in_specs=[
            pl.BlockSpec((ROW_BLK, D), lambda i: (i, 0)),
            full,
            pl.BlockSpec((1, H), lambda i: (0, 0)),
            full, full, full,
        ],
        out_specs=[blk, blk, blk, blk],
        out_shape=[jax.ShapeDtypeStruct((N, H), jnp.float32)] * 4,
    )(x, l1w_t, l1b, w1_t, w2_t, wf_t)


def _stage_b_body(hf_ref, h4_ref, agg_ref, w1_ref, w2_ref, wf_ref,
                  h3_ref, h4o_ref, hfo_ref):
    g = agg_ref[0, :, :] + agg_ref[1, :, :]
    v = hf_ref[...] + STEP * g - STEP * h4_ref[...]
    nrm = jnp.sqrt(jnp.sum(v * v, axis=1, keepdims=True))
    q = v / jnp.maximum(nrm, 1e-12)
    h3_ref[...] = jnp.dot(q, w1_ref[...], preferred_element_type=jnp.float32)
    h4o_ref[...] = jnp.dot(q, w2_ref[...], preferred_element_type=jnp.float32)
    hfo_ref[...] = jnp.dot(q, wf_ref[...], preferred_element_type=jnp.float32)


def _stage_b(hf, h4, agg, w1_t, w2_t, wf_t):
    grid = (N // ROW_BLK,)
    full = pl.BlockSpec((H, H), lambda i: (0, 0))
    blk = pl.BlockSpec((ROW_BLK, H), lambda i: (i, 0))
    return pl.pallas_call(
        _stage_b_body,
        grid=grid,
        in_specs=[
            blk, blk,
            pl.BlockSpec((2, ROW_BLK, H), lambda i: (0, i, 0)),
            full, full, full,
        ],
        out_specs=[blk, blk, blk],
        out_shape=[jax.ShapeDtypeStruct((N, H), jnp.float32)] * 3,
    )(hf, h4, agg, w1_t, w2_t, wf_t)


def _stage_c_body(hf_ref, h4_ref, agg_ref, q2_ref, wa_ref, wb_ref, b_ref,
                  out_ref):
    g = agg_ref[0, :, :] + agg_ref[1, :, :]
    v = hf_ref[...] + STEP * g - STEP * h4_ref[...]
    nrm = jnp.sqrt(jnp.sum(v * v, axis=1, keepdims=True))
    q3 = v / jnp.maximum(nrm, 1e-12)
    logits = (jnp.dot(q2_ref[...], wa_ref[...], preferred_element_type=jnp.float32)
              + jnp.dot(q3, wb_ref[...], preferred_element_type=jnp.float32)
              + b_ref[...])
    m = jnp.max(logits, axis=1, keepdims=True)
    ex = jnp.exp(logits - m)
    lse = jnp.log(jnp.sum(ex, axis=1, keepdims=True))
    out_ref[...] = logits - m - lse


def _stage_c(hf, h4, agg, q2, wa_t, wb_t, b_pad):
    grid = (N // ROW_BLK,)
    full = pl.BlockSpec((H, CPAD), lambda i: (0, 0))
    blk = pl.BlockSpec((ROW_BLK, H), lambda i: (i, 0))
    return pl.pallas_call(
        _stage_c_body,
        grid=grid,
        in_specs=[
            blk, blk,
            pl.BlockSpec((2, ROW_BLK, H), lambda i: (0, i, 0)),
            blk,
            full, full,
            pl.BlockSpec((1, CPAD), lambda i: (0, 0)),
        ],
        out_specs=pl.BlockSpec((ROW_BLK, CPAD), lambda i: (i, 0)),
        out_shape=jax.ShapeDtypeStruct((N, CPAD), jnp.float32),
    )(hf, h4, agg, q2, wa_t, wb_t, b_pad)


def _edge_body(h3_hbm, h4_hbm, row_hbm, col_hbm, out_hbm,
               row_b, col_b, h3r_b, h4r_b, agg_sh, sem_i, sem_g, sem_s):
    c = lax.axis_index("c")
    s = lax.axis_index("s")
    wid = c * 16 + s

    def zero_body(i, carry):
        for t in range(8):
            h3r_b[0][i, pl.ds(t * 16, 16)] = jnp.zeros((16,), jnp.float32)
        return carry

    lax.fori_loop(0, CHUNK, zero_body, 0)
    for d in range(RPT // CHUNK):
        pltpu.sync_copy(h3r_b[0], agg_sh.at[pl.ds(s * RPT + d * CHUNK, CHUNK)])
    plsc.subcore_barrier()

    base = wid * EPT

    def idx_start(j, bi):
        off = base + j * CHUNK
        pltpu.async_copy(row_hbm.at[pl.ds(off, CHUNK)], row_b[bi], sem_i[bi])
        pltpu.async_copy(col_hbm.at[pl.ds(off, CHUNK)], col_b[bi], sem_i[bi])

    def idx_wait(j, bi):
        off = base + j * CHUNK
        pltpu.make_async_copy(row_hbm.at[pl.ds(off, CHUNK)], row_b[bi],
                              sem_i[bi]).wait()
        pltpu.make_async_copy(col_hbm.at[pl.ds(off, CHUNK)], col_b[bi],
                              sem_i[bi]).wait()

    def gather_start(bb, bi):
        pltpu.async_copy(h3_hbm.at[row_b[bi]], h3r_b[bb], sem_g[bb])
        pltpu.async_copy(h4_hbm.at[col_b[bi]], h4r_b[bb], sem_g[bb])

    def gather_wait(bb, bi):
        pltpu.make_async_copy(h3_hbm.at[row_b[bi]], h3r_b[bb],
                              sem_g[bb]).wait()
        pltpu.make_async_copy(h4_hbm.at[col_b[bi]], h4r_b[bb],
                              sem_g[bb]).wait()

    def scatter_wait(bb, bi):
        pltpu.make_async_copy(h3r_b[bb], agg_sh.at[col_b[bi]],
                              sem_s[bb]).wait()

    def compute(bb, bi):
        h3r_v, h4r_v = h3r_b[bb], h4r_b[bb]
        gather_wait(bb, bi)

        @plsc.parallel_loop(0, CHUNK, unroll=2)
        def edge_body(e):
            v3 = [h3r_v[e, pl.ds(t * 16, 16)] for t in range(8)]
            acc = v3[0] * h4r_v[e, pl.ds(0, 16)]
            for t in range(1, 8):
                acc = acc + v3[t] * h4r_v[e, pl.ds(t * 16, 16)]
            dot = jnp.sum(acc)
            sv = 1.0 / (1.0 + jnp.exp(jnp.broadcast_to(dot, (16,))))
            for t in range(8):
                h3r_v[e, pl.ds(t * 16, 16)] = v3[t] * sv

        pltpu.async_copy(h3r_v, agg_sh.at[col_b[bi]], sem_s[bb], add=True)

    idx_start(0, 0)
    idx_start(1, 1)
    idx_start(2, 2)
    idx_wait(0, 0)
    gather_start(0, 0)
    idx_wait(1, 1)
    gather_start(1, 1)

    def oct_body(p, carry):
        for b8 in range(NIDX):
            j = NIDX * p + b8
            bb = b8 % NBUF

            @pl.when(jnp.logical_and(j - 2 >= 0, j - 2 < NCHUNK))
            def _():
                scatter_wait((b8 - 2) % NBUF, (b8 - 2) % NIDX)

            @pl.when(j + 3 < NCHUNK)
            def _():
                idx_start(j + 3, (b8 + 3) % NIDX)

            @pl.when(j + 2 < NCHUNK)
            def _():
                idx_wait(j + 2, (b8 + 2) % NIDX)
                gather_start((b8 + 2) % NBUF, (b8 + 2) % NIDX)

            @pl.when(j < NCHUNK)
            def _():
                compute(bb, b8)

        return carry

    lax.fori_loop(0, (NCHUNK + NIDX - 1) // NIDX + 1, oct_body, 0)
    plsc.subcore_barrier()

    pltpu.sync_copy(agg_sh.at[pl.ds(s * RPT, RPT)],
                    out_hbm.at[c, pl.ds(s * RPT, RPT)])


@functools.cache
def _edge_kernel():
    return pl.kernel(
        _edge_body,
        mesh=plsc.VectorSubcoreMesh(core_axis_name="c", subcore_axis_name="s"),
        out_type=jax.ShapeDtypeStruct((2, NPAD, H), jnp.float32),
        scratch_types=[
            [pltpu.VMEM((CHUNK,), jnp.int32)] * NIDX,
            [pltpu.VMEM((CHUNK,), jnp.int32)] * NIDX,
            [pltpu.VMEM((CHUNK, H), jnp.float32)] * NBUF,
            [pltpu.VMEM((CHUNK, H), jnp.float32)] * NBUF,
            pltpu.VMEM_SHARED((NPAD, H), jnp.float32),
            [pltpu.SemaphoreType.DMA] * NIDX,
            [pltpu.SemaphoreType.DMA] * NBUF,
            [pltpu.SemaphoreType.DMA] * NBUF,
        ],
        compiler_params=pltpu.CompilerParams(needs_layout_passes=False),
    )


def kernel(x, edge_index, lin1_W, lin1_b, W1_0, W2_0, Wf_0, W1_1, W2_1, Wf_1,
           out_W, out_b):
    row = edge_index[0]
    col = edge_index[1]

    q2, h3, h4, hf = _stage_a(
        x, lin1_W.T, lin1_b.reshape(1, H), W1_0.T, W2_0.T, Wf_0.T)
    agg0 = _edge_kernel()(h3, h4, row, col)
    h3, h4, hf = _stage_b(hf, h4, agg0, W1_1.T, W2_1.T, Wf_1.T)
    agg1 = _edge_kernel()(h3, h4, row, col)

    wa_t = jnp.zeros((H, CPAD), jnp.float32).at[:, :C].set(out_W[:, :H].T)
    wb_t = jnp.zeros((H, CPAD), jnp.float32).at[:, :C].set(out_W[:, H:].T)
    b_pad = jnp.full((1, CPAD), -1e30, jnp.float32).at[0, :C].set(out_b)

    out_pad = _stage_c(hf, h4, agg1, q2, wa_t, wb_t, b_pad)
    return out_pad[:, :C]


# speedup vs baseline: 1.2429x; 1.0013x over previous
import functools

import jax
import jax.numpy as jnp
from jax import lax
from jax.experimental import pallas as pl
from jax.experimental.pallas import tpu as pltpu
from jax.experimental.pallas import tpu_sc as plsc

N = 10000
E = 320000
D = 128
H = 128
C = 40
STEP = 0.1
CPAD = 128

ROW_BLK = 1000

NUM_TILES = 32
EPT = E // NUM_TILES
CHUNK = 40
NCHUNK = EPT // CHUNK
NB3 = 5
NB4 = 4
NIDX = 10
NSEC = 20
NPAD = 10240
RPT = NPAD // 16


def _stage_a_body(x_ref, l1w_ref, l1b_ref, w1_ref, w2_ref, wf_ref,
                  q_ref, h3_ref, h4_ref, hf_ref):
    q = jnp.dot(x_ref[...], l1w_ref[...], preferred_element_type=jnp.float32)
    q = jnp.maximum(q + l1b_ref[...], 0.0)
    q_ref[...] = q
    h3_ref[...] = jnp.dot(q, w1_ref[...], preferred_element_type=jnp.float32)
    h4_ref[...] = jnp.dot(q, w2_ref[...], preferred_element_type=jnp.float32)
    hf_ref[...] = jnp.dot(q, wf_ref[...], preferred_element_type=jnp.float32)


def _stage_a(x, l1w_t, l1b, w1_t, w2_t, wf_t):
    grid = (N // ROW_BLK,)
    full = pl.BlockSpec((D, H), lambda i: (0, 0))
    blk = pl.BlockSpec((ROW_BLK, H), lambda i: (i, 0))
    return pl.pallas_call(
        _stage_a_body,
        grid=grid,
        in_specs=[
            pl.BlockSpec((ROW_BLK, D), lambda i: (i, 0)),
            full,
            pl.BlockSpec((1, H), lambda i: (0, 0)),
            full, full, full,
        ],
        out_specs=[blk, blk, blk, blk],
        out_shape=[jax.ShapeDtypeStruct((N, H), jnp.float32)] * 4,
    )(x, l1w_t, l1b, w1_t, w2_t, wf_t)


def _stage_b_body(hf_ref, h4_ref, agg_ref, w1_ref, w2_ref, wf_ref,
                  h3_ref, h4o_ref, hfo_ref):
    g = agg_ref[0, :, :] + agg_ref[1, :, :]
    v = hf_ref[...] + STEP * g - STEP * h4_ref[...]
    nrm = jnp.sqrt(jnp.sum(v * v, axis=1, keepdims=True))
    q = v / jnp.maximum(nrm, 1e-12)
    h3_ref[...] = jnp.dot(q, w1_ref[...], preferred_element_type=jnp.float32)
    h4o_ref[...] = jnp.dot(q, w2_ref[...], preferred_element_type=jnp.float32)
    hfo_ref[...] = jnp.dot(q, wf_ref[...], preferred_element_type=jnp.float32)


def _stage_b(hf, h4, agg, w1_t, w2_t, wf_t):
    grid = (N // ROW_BLK,)
    full = pl.BlockSpec((H, H), lambda i: (0, 0))
    blk = pl.BlockSpec((ROW_BLK, H), lambda i: (i, 0))
    return pl.pallas_call(
        _stage_b_body,
        grid=grid,
        in_specs=[
            blk, blk,
            pl.BlockSpec((2, ROW_BLK, H), lambda i: (0, i, 0)),
            full, full, full,
        ],
        out_specs=[blk, blk, blk],
        out_shape=[jax.ShapeDtypeStruct((N, H), jnp.float32)] * 3,
    )(hf, h4, agg, w1_t, w2_t, wf_t)


def _stage_c_body(hf_ref, h4_ref, agg_ref, q2_ref, wa_ref, wb_ref, b_ref,
                  out_ref):
    g = agg_ref[0, :, :] + agg_ref[1, :, :]
    v = hf_ref[...] + STEP * g - STEP * h4_ref[...]
    nrm = jnp.sqrt(jnp.sum(v * v, axis=1, keepdims=True))
    q3 = v / jnp.maximum(nrm, 1e-12)
    logits = (jnp.dot(q2_ref[...], wa_ref[...], preferred_element_type=jnp.float32)
              + jnp.dot(q3, wb_ref[...], preferred_element_type=jnp.float32)
              + b_ref[...])
    m = jnp.max(logits, axis=1, keepdims=True)
    ex = jnp.exp(logits - m)
    lse = jnp.log(jnp.sum(ex, axis=1, keepdims=True))
    out_ref[...] = logits - m - lse


def _stage_c(hf, h4, agg, q2, wa_t, wb_t, b_pad):
    grid = (N // ROW_BLK,)
    full = pl.BlockSpec((H, CPAD), lambda i: (0, 0))
    blk = pl.BlockSpec((ROW_BLK, H), lambda i: (i, 0))
    return pl.pallas_call(
        _stage_c_body,
        grid=grid,
        in_specs=[
            blk, blk,
            pl.BlockSpec((2, ROW_BLK, H), lambda i: (0, i, 0)),
            blk,
            full, full,
            pl.BlockSpec((1, CPAD), lambda i: (0, 0)),
        ],
        out_specs=pl.BlockSpec((ROW_BLK, CPAD), lambda i: (i, 0)),
        out_shape=jax.ShapeDtypeStruct((N, CPAD), jnp.float32),
    )(hf, h4, agg, q2, wa_t, wb_t, b_pad)


def _edge_body(h3_hbm, h4_hbm, row_hbm, col_hbm, out_hbm,
               row_b, col_b, h3r_b, h4r_b, agg_sh, sem_i, sem_g3, sem_g4,
               sem_s):
    c = lax.axis_index("c")
    s = lax.axis_index("s")
    wid = c * 16 + s

    def zero_body(i, carry):
        for t in range(8):
            h3r_b[0][i, pl.ds(t * 16, 16)] = jnp.zeros((16,), jnp.float32)
        return carry

    lax.fori_loop(0, CHUNK, zero_body, 0)
    for d in range(RPT // CHUNK):
        pltpu.sync_copy(h3r_b[0], agg_sh.at[pl.ds(s * RPT + d * CHUNK, CHUNK)])
    plsc.subcore_barrier()

    base = wid * EPT

    def idx_start(j, bi):
        off = base + j * CHUNK
        pltpu.async_copy(row_hbm.at[pl.ds(off, CHUNK)], row_b[bi], sem_i[bi])
        pltpu.async_copy(col_hbm.at[pl.ds(off, CHUNK)], col_b[bi], sem_i[bi])

    def idx_wait(j, bi):
        off = base + j * CHUNK
        pltpu.make_async_copy(row_hbm.at[pl.ds(off, CHUNK)], row_b[bi],
                              sem_i[bi]).wait()
        pltpu.make_async_copy(col_hbm.at[pl.ds(off, CHUNK)], col_b[bi],
                              sem_i[bi]).wait()

    def gather_start(b3, b4, bi):
        pltpu.async_copy(h3_hbm.at[row_b[bi]], h3r_b[b3], sem_g3[b3])
        pltpu.async_copy(h4_hbm.at[col_b[bi]], h4r_b[b4], sem_g4[b4])

    def gather_wait(b3, b4, bi):
        pltpu.make_async_copy(h3_hbm.at[row_b[bi]], h3r_b[b3],
                              sem_g3[b3]).wait()
        pltpu.make_async_copy(h4_hbm.at[col_b[bi]], h4r_b[b4],
                              sem_g4[b4]).wait()

    def scatter_wait(b3, bi):
        pltpu.make_async_copy(h3r_b[b3], agg_sh.at[col_b[bi]],
                              sem_s[b3]).wait()

    def compute(b3, b4, bi):
        h3r_v, h4r_v = h3r_b[b3], h4r_b[b4]
        gather_wait(b3, b4, bi)

        @plsc.parallel_loop(0, CHUNK, unroll=2)
        def edge_body(e):
            v3 = [h3r_v[e, pl.ds(t * 16, 16)] for t in range(8)]
            acc = v3[0] * h4r_v[e, pl.ds(0, 16)]
            for t in range(1, 8):
                acc = acc + v3[t] * h4r_v[e, pl.ds(t * 16, 16)]
            dot = jnp.sum(acc)
            sv = 1.0 / (1.0 + jnp.exp(jnp.broadcast_to(dot, (16,))))
            for t in range(8):
                h3r_v[e, pl.ds(t * 16, 16)] = v3[t] * sv

        pltpu.async_copy(h3r_v, agg_sh.at[col_b[bi]], sem_s[b3], add=True)

    for jj in range(4):
        idx_start(jj, jj)
    for jj in range(3):
        idx_wait(jj, jj)
        gather_start(jj % NB3, jj % NB4, jj % NIDX)

    def sec_body(p, carry):
        for b in range(NSEC):
            j = NSEC * p + b

            @pl.when(jnp.logical_and(j - 2 >= 0, j - 2 < NCHUNK))
            def _():
                scatter_wait((b - 2) % NB3, (b - 2) % NIDX)

            @pl.when(j + 4 < NCHUNK)
            def _():
                idx_start(j + 4, (b + 4) % NIDX)

            @pl.when(j + 3 < NCHUNK)
            def _():
                idx_wait(j + 3, (b + 3) % NIDX)
                gather_start((b + 3) % NB3, (b + 3) % NB4, (b + 3) % NIDX)

            @pl.when(j < NCHUNK)
            def _():
                compute(b % NB3, b % NB4, b % NIDX)

        return carry

    lax.fori_loop(0, (NCHUNK + NSEC - 1) // NSEC + 1, sec_body, 0)
    plsc.subcore_barrier()

    pltpu.sync_copy(agg_sh.at[pl.ds(s * RPT, RPT)],
                    out_hbm.at[c, pl.ds(s * RPT, RPT)])


@functools.cache
def _edge_kernel():
    return pl.kernel(
        _edge_body,
        mesh=plsc.VectorSubcoreMesh(core_axis_name="c", subcore_axis_name="s"),
        out_type=jax.ShapeDtypeStruct((2, NPAD, H), jnp.float32),
        scratch_types=[
            [pltpu.VMEM((CHUNK,), jnp.int32)] * NIDX,
            [pltpu.VMEM((CHUNK,), jnp.int32)] * NIDX,
            [pltpu.VMEM((CHUNK, H), jnp.float32)] * NB3,
            [pltpu.VMEM((CHUNK, H), jnp.float32)] * NB4,
            pltpu.VMEM_SHARED((NPAD, H), jnp.float32),
            [pltpu.SemaphoreType.DMA] * NIDX,
            [pltpu.SemaphoreType.DMA] * NB3,
            [pltpu.SemaphoreType.DMA] * NB4,
            [pltpu.SemaphoreType.DMA] * NB3,
        ],
        compiler_params=pltpu.CompilerParams(needs_layout_passes=False),
    )


def kernel(x, edge_index, lin1_W, lin1_b, W1_0, W2_0, Wf_0, W1_1, W2_1, Wf_1,
           out_W, out_b):
    row = edge_index[0]
    col = edge_index[1]

    q2, h3, h4, hf = _stage_a(
        x, lin1_W.T, lin1_b.reshape(1, H), W1_0.T, W2_0.T, Wf_0.T)
    agg0 = _edge_kernel()(h3, h4, row, col)
    h3, h4, hf = _stage_b(hf, h4, agg0, W1_1.T, W2_1.T, Wf_1.T)
    agg1 = _edge_kernel()(h3, h4, row, col)

    wa_t = jnp.zeros((H, CPAD), jnp.float32).at[:, :C].set(out_W[:, :H].T)
    wb_t = jnp.zeros((H, CPAD), jnp.float32).at[:, :C].set(out_W[:, H:].T)
    b_pad = jnp.full((1, CPAD), -1e30, jnp.float32).at[0, :C].set(out_b)

    out_pad = _stage_c(hf, h4, agg1, q2, wa_t, wb_t, b_pad)
    return out_pad[:, :C]
